# baseline (speedup 1.0000x reference)
import functools

import jax
import jax.numpy as jnp
from jax import lax
from jax.experimental import pallas as pl
from jax.experimental.pallas import tpu as pltpu
from jax.experimental.pallas import tpu_sc as plsc


@functools.lru_cache(maxsize=None)
def _make_rel_pos_kernel(q_len, v_len, in_dim, out_dim):
    info = plsc.get_sparse_core_info()
    nc, ns = info.num_cores, info.num_subcores
    nw = nc * ns

    mp = (in_dim - 1) // 2
    pre = q_len - 1 - mp
    suf_start = pre + in_dim
    r_len = q_len + v_len - 1
    assert suf_start + (v_len - 1 - mp) == r_len
    assert v_len - 1 - mp == pre
    assert q_len % nw == 0 and out_dim % 16 == 0
    rows_per_w = q_len // nw

    mesh = plsc.VectorSubcoreMesh(core_axis_name="c", subcore_axis_name="s")

    @functools.partial(
        pl.kernel,
        out_type=jax.ShapeDtypeStruct((q_len, v_len, out_dim), jnp.float32),
        mesh=mesh,
        scratch_types=[
            pltpu.VMEM((r_len, out_dim), jnp.float32),
            pltpu.SemaphoreType.DMA,
        ],
        compiler_params=pltpu.CompilerParams(use_tc_tiling_on_sc=False),
    )
    def rel_pos(emb_hbm, out_hbm, r_v, sem):
        wid = lax.axis_index("s") * nc + lax.axis_index("c")

        pltpu.sync_copy(emb_hbm, r_v.at[pl.ds(pre, in_dim)])
        nchunk = out_dim // 16
        first = [r_v[pre, pl.ds(c * 16, 16)] for c in range(nchunk)]
        last = [r_v[suf_start - 1, pl.ds(c * 16, 16)] for c in range(nchunk)]

        def fill(t, carry):
            for c in range(nchunk):
                r_v[t, pl.ds(c * 16, 16)] = first[c]
                r_v[suf_start + t, pl.ds(c * 16, 16)] = last[c]
            return carry

        lax.fori_loop(0, pre, fill, 0)

        base = wid * rows_per_w
        handles = []
        for r in range(rows_per_w):
            i = base + r
            src = r_v.at[pl.ds(q_len - 1 - i, v_len)]
            handles.append(pltpu.async_copy(src, out_hbm.at[i], sem))
        for h in handles:
            h.wait()

    return rel_pos


def kernel(q, v, embeddings):
    q_len = q.shape[1]
    v_len = v.shape[1]
    in_dim, out_dim = embeddings.shape
    rel_pos = _make_rel_pos_kernel(q_len, v_len, in_dim, out_dim)
    return rel_pos(embeddings)


# speedup vs baseline: 2.7507x; 2.7507x over previous
import functools

import jax
import jax.numpy as jnp
from jax import lax
from jax.experimental import pallas as pl
from jax.experimental.pallas import tpu as pltpu
from jax.experimental.pallas import tpu_sc as plsc


@functools.lru_cache(maxsize=None)
def _make_rel_pos_kernel(q_len, v_len, in_dim, out_dim):
    info = plsc.get_sparse_core_info()
    nc, ns = info.num_cores, info.num_subcores
    nw = nc * ns

    mp = (in_dim - 1) // 2
    pre = q_len - 1 - mp
    suf_start = pre + in_dim
    r_len = q_len + v_len - 1
    assert suf_start + (v_len - 1 - mp) == r_len
    assert q_len % nw == 0 and nw % 8 == 0 and in_dim % 8 == 1
    rows_per_w = q_len // nw

    wd = r_len - 7
    assert wd % 8 == 0 and wd <= 131071 // out_dim
    ph_w = in_dim + 7
    pre_end = pre + 1
    suf_fill = ((suf_start - 7) // 8) * 8
    assert pre_end % 8 == 0 and pre_end >= 16 and suf_fill + 8 >= suf_start - 7
    assert wd % 8 == 0 and (wd - suf_fill) % 8 == 0

    mesh = plsc.VectorSubcoreMesh(core_axis_name="c", subcore_axis_name="s")

    @functools.partial(
        pl.kernel,
        out_type=jax.ShapeDtypeStruct((q_len, out_dim, v_len), jnp.float32),
        mesh=mesh,
        scratch_types=[
            pltpu.VMEM((out_dim, wd), jnp.float32),
            pltpu.SemaphoreType.DMA,
        ],
        compiler_params=pltpu.CompilerParams(use_tc_tiling_on_sc=False),
    )
    def rel_pos(phases_hbm, pref_hbm, suf_hbm, out_hbm, w_v, sem):
        wid = lax.axis_index("s") * nc + lax.axis_index("c")
        off = lax.rem(jnp.int32(q_len - 1) - wid, jnp.int32(8))
        u_b = jnp.int32(pre) - off
        delta = lax.rem(u_b, jnp.int32(8))
        a = pl.multiple_of(u_b - delta, 8)

        pltpu.sync_copy(pref_hbm, w_v.at[:, pl.ds(0, pre_end)])
        pltpu.sync_copy(suf_hbm, w_v.at[:, pl.ds(suf_fill, wd - suf_fill)])
        pltpu.sync_copy(phases_hbm.at[delta], w_v.at[:, pl.ds(a, ph_w)])

        handles = []
        for t in range(rows_per_w):
            i = wid + t * nw
            u0 = pl.multiple_of(jnp.int32(q_len - 1) - i - off, 8)
            src = w_v.at[:, pl.ds(u0, v_len)]
            handles.append(pltpu.async_copy(src, out_hbm.at[i], sem))
        for h in handles:
            h.wait()

    return rel_pos


def kernel(q, v, embeddings):
    q_len = q.shape[1]
    v_len = v.shape[1]
    in_dim, out_dim = embeddings.shape
    rel_pos = _make_rel_pos_kernel(q_len, v_len, in_dim, out_dim)

    emb_t = embeddings.T
    first = emb_t[:, :1]
    last = emb_t[:, -1:]
    ph_w = in_dim + 7
    phases = jnp.stack([
        jnp.concatenate(
            [jnp.broadcast_to(first, (out_dim, p)), emb_t,
             jnp.broadcast_to(last, (out_dim, ph_w - in_dim - p))], axis=1)
        for p in range(8)
    ])
    mp = (in_dim - 1) // 2
    pre_end = q_len - mp
    r_len = q_len + v_len - 1
    wd = r_len - 7
    suf_fill = ((pre_end - 1 + in_dim - 7) // 8) * 8
    pref_tpl = jnp.broadcast_to(first, (out_dim, pre_end))
    suf_tpl = jnp.broadcast_to(last, (out_dim, wd - suf_fill))

    out_t = rel_pos(phases, pref_tpl, suf_tpl)
    return out_t.transpose(0, 2, 1)


# speedup vs baseline: 9.6200x; 3.4972x over previous
import functools

import jax
import jax.numpy as jnp
from jax import lax
from jax.experimental import pallas as pl
from jax.experimental.pallas import tpu as pltpu
from jax.experimental.pallas import tpu_sc as plsc


@functools.lru_cache(maxsize=None)
def _make_rel_pos_kernel(q_len, v_len, in_dim, out_dim):
    info = plsc.get_sparse_core_info()
    nc, ns = info.num_cores, info.num_subcores
    nw = nc * ns

    mp = (in_dim - 1) // 2
    pre = q_len - 1 - mp
    suf_start = pre + in_dim
    r_len = q_len + v_len - 1
    assert suf_start + (v_len - 1 - mp) == r_len
    assert q_len % nw == 0 and nw % 8 == 0 and in_dim % 8 == 1
    assert out_dim % 8 == 0 and v_len % 128 == 0
    rows_per_w = q_len // nw
    n_tk = out_dim // 8
    n_tj = v_len // 128

    wd = r_len - 7
    assert wd % 8 == 0 and wd <= 131071 // out_dim
    ph_w = in_dim + 7
    pre_end = pre + 1
    suf_fill = ((suf_start - 7) // 8) * 8
    assert pre_end % 8 == 0 and pre_end >= 16 and suf_fill + 8 >= suf_start - 7
    assert wd % 8 == 0 and (wd - suf_fill) % 8 == 0

    mesh = plsc.VectorSubcoreMesh(core_axis_name="c", subcore_axis_name="s")

    @functools.partial(
        pl.kernel,
        out_type=jax.ShapeDtypeStruct((q_len, n_tk, n_tj, 8, 128),
                                      jnp.float32),
        mesh=mesh,
        scratch_types=[
            pltpu.VMEM((out_dim, wd), jnp.float32),
            pltpu.SemaphoreType.DMA,
        ],
        compiler_params=pltpu.CompilerParams(use_tc_tiling_on_sc=False),
    )
    def rel_pos(phases_hbm, pref_hbm, suf_hbm, out_hbm, w_v, sem):
        wid = lax.axis_index("s") * nc + lax.axis_index("c")
        off = lax.rem(jnp.int32(q_len - 1) - wid, jnp.int32(8))
        u_b = jnp.int32(pre) - off
        delta = lax.rem(u_b, jnp.int32(8))
        a = pl.multiple_of(u_b - delta, 8)

        pltpu.sync_copy(pref_hbm, w_v.at[:, pl.ds(0, pre_end)])
        pltpu.sync_copy(suf_hbm, w_v.at[:, pl.ds(suf_fill, wd - suf_fill)])
        pltpu.sync_copy(phases_hbm.at[delta], w_v.at[:, pl.ds(a, ph_w)])

        def fire(t, carry):
            i = wid + t * nw
            u0 = pl.multiple_of(jnp.int32(q_len - 1) - i - off, 8)
            handles = []
            for tk in range(n_tk):
                for tj in range(n_tj):
                    src = w_v.at[pl.ds(8 * tk, 8),
                                 pl.ds(pl.multiple_of(u0 + 128 * tj, 8), 128)]
                    handles.append(
                        pltpu.async_copy(src, out_hbm.at[i, tk, tj], sem))
            for h in handles:
                h.wait()
            return carry

        lax.fori_loop(0, rows_per_w, fire, 0)

    return rel_pos


def kernel(q, v, embeddings):
    q_len = q.shape[1]
    v_len = v.shape[1]
    in_dim, out_dim = embeddings.shape
    rel_pos = _make_rel_pos_kernel(q_len, v_len, in_dim, out_dim)

    emb_t = embeddings.T
    first = emb_t[:, :1]
    last = emb_t[:, -1:]
    ph_w = in_dim + 7
    phases = jnp.stack([
        jnp.concatenate(
            [jnp.broadcast_to(first, (out_dim, p)), emb_t,
             jnp.broadcast_to(last, (out_dim, ph_w - in_dim - p))], axis=1)
        for p in range(8)
    ])
    mp = (in_dim - 1) // 2
    pre_end = q_len - mp
    r_len = q_len + v_len - 1
    wd = r_len - 7
    suf_fill = ((pre_end - 1 + in_dim - 7) // 8) * 8
    pref_tpl = jnp.broadcast_to(first, (out_dim, pre_end))
    suf_tpl = jnp.broadcast_to(last, (out_dim, wd - suf_fill))

    f5 = rel_pos(phases, pref_tpl, suf_tpl)
    out_t = f5.transpose(0, 1, 3, 2, 4).reshape(q_len, out_dim, v_len)
    return out_t.transpose(0, 2, 1)
